# baseline (speedup 1.0000x reference)
import functools

import jax
import jax.numpy as jnp
from jax import lax
from jax.experimental import pallas as pl
from jax.experimental.pallas import tpu as pltpu
from jax.experimental.pallas import tpu_sc as plsc

_SIZE = 65536
_VALID = 40001
_NS = 16
_CHUNK = _SIZE // _NS
_NVEC = _CHUNK // 16
_L = 16


def _sc_body(hist, params, pidx, out, thr, parts, buf, par_v, idx_v, stage,
             all_v, thr_v, sem):
    w = lax.axis_index("s")
    base = w * _CHUNK
    pltpu.sync_copy(params, par_v)
    pltpu.sync_copy(pidx, idx_v)
    pltpu.sync_copy(hist.at[pl.ds(base, _CHUNK)], buf)

    nv = par_v[0]
    halfnoise = par_v[1]
    idx = idx_v[...]
    lane = lax.broadcasted_iota(jnp.int32, (_L,), 0)
    lidx = idx - base
    hit = (lidx >= 0) & (lidx < _CHUNK) & (lane == 0)
    plsc.store_scatter(buf, [lidx], nv, mask=hit)

    cp = pltpu.async_copy(buf, out.at[pl.ds(base, _CHUNK)], sem)

    nrem = jnp.maximum(_VALID - base, 0)
    nfull = jnp.minimum(nrem // _L, _NVEC)
    zero = jnp.zeros((_L,), jnp.float32)

    def step(j, carry):
        s, q = carry
        v = buf[pl.ds(j * _L, _L)]
        return s + v, q + v * v

    s, q = lax.fori_loop(0, nfull, step, (zero, zero))

    tail = (nrem - nfull * _L > 0) & (nfull < _NVEC)

    @pl.when(tail)
    def _():
        v = buf[pl.ds(nfull * _L, _L)]
        vm = jnp.where(lane < nrem - nfull * _L, v, 0.0)
        stage[0] = s + vm
        stage[1] = q + vm * vm

    @pl.when(jnp.logical_not(tail))
    def _():
        stage[0] = s
        stage[1] = q

    pltpu.sync_copy(stage, parts.at[w])
    plsc.subcore_barrier()

    @pl.when(w == 0)
    def _():
        pltpu.sync_copy(parts, all_v)
        S = zero
        Q = zero
        for i in range(_NS):
            S = S + all_v[i, 0]
            Q = Q + all_v[i, 1]
        ssum = jnp.sum(S)
        qsum = jnp.sum(Q)
        inv_n = jnp.float32(1.0 / _VALID)
        mean = ssum * inv_n
        var = jnp.maximum(qsum * inv_n - mean * mean, 0.0)
        varv = zero + var
        x = 0.5 * (varv + 1.0)
        for _ in range(10):
            x = 0.5 * (x + varv / x)
        stdv = jnp.where(varv > 0.0, x, 0.0)
        thr_v[...] = (zero + mean) + halfnoise * stdv
        pltpu.sync_copy(thr_v, thr)

    cp.wait()


_sc_call = pl.kernel(
    _sc_body,
    out_type=(
        jax.ShapeDtypeStruct((_SIZE,), jnp.float32),
        jax.ShapeDtypeStruct((_L,), jnp.float32),
        jax.ShapeDtypeStruct((_NS, 2, _L), jnp.float32),
    ),
    mesh=plsc.VectorSubcoreMesh(
        core_axis_name="c", subcore_axis_name="s", num_cores=1,
        num_subcores=_NS),
    compiler_params=pltpu.CompilerParams(needs_layout_passes=False),
    scratch_types=[
        pltpu.VMEM((_CHUNK,), jnp.float32),
        pltpu.VMEM((2, _L), jnp.float32),
        pltpu.VMEM((_L,), jnp.int32),
        pltpu.VMEM((2, _L), jnp.float32),
        pltpu.VMEM((_NS, 2, _L), jnp.float32),
        pltpu.VMEM((_L,), jnp.float32),
        pltpu.SemaphoreType.DMA,
    ],
)


@jax.jit
def kernel(history, new_value, pointer):
    idx = jnp.asarray(pointer, jnp.int32) % _SIZE
    noise = jax.random.normal(jax.random.key(42), (), dtype=jnp.float32)
    params = jnp.stack([
        jnp.full((_L,), jnp.asarray(new_value, jnp.float32)),
        jnp.full((_L,), noise * jnp.float32(0.5)),
    ])
    pidx = jnp.full((_L,), idx, jnp.int32)
    updated, thr, _ = _sc_call(history, params, pidx)
    return updated, thr[0]


# speedup vs baseline: 5.4177x; 5.4177x over previous
import jax
import jax.numpy as jnp
from jax import lax
from jax.experimental import pallas as pl
from jax.experimental.pallas import tpu as pltpu

_SIZE = 65536
_VALID = 40001
_ROWS = 512
_COLS = 128

def _body(scal_ref, hist_ref, out_ref, thr_ref):
    h = hist_ref[...]
    idx = scal_ref[0, 0]
    nv = lax.bitcast_convert_type(scal_ref[0, 1], jnp.float32)
    halfnoise = lax.bitcast_convert_type(scal_ref[0, 2], jnp.float32)
    r = idx // _COLS
    c = idx % _COLS
    rows = lax.broadcasted_iota(jnp.int32, (_ROWS, _COLS), 0)
    cols = lax.broadcasted_iota(jnp.int32, (_ROWS, _COLS), 1)
    upd = jnp.where((rows == r) & (cols == c), nv, h)
    out_ref[...] = upd
    flat = rows * _COLS + cols
    vm = jnp.where(flat < _VALID, upd, 0.0)
    s = jnp.sum(vm)
    q = jnp.sum(vm * vm)
    inv_n = jnp.float32(1.0 / _VALID)
    mean = s * inv_n
    var = jnp.maximum(q * inv_n - mean * mean, 0.0)
    std = jnp.sqrt(var)
    thr_ref[0, 0] = mean + halfnoise * std


_call = pl.pallas_call(
    _body,
    out_shape=(
        jax.ShapeDtypeStruct((_ROWS, _COLS), jnp.float32),
        jax.ShapeDtypeStruct((1, 1), jnp.float32),
    ),
    in_specs=[
        pl.BlockSpec(memory_space=pltpu.SMEM),
        pl.BlockSpec(memory_space=pltpu.VMEM),
    ],
    out_specs=(
        pl.BlockSpec(memory_space=pltpu.VMEM),
        pl.BlockSpec(memory_space=pltpu.SMEM),
    ),
)


@jax.jit
def kernel(history, new_value, pointer):
    idx = jnp.asarray(pointer, jnp.int32) % _SIZE
    nv_bits = lax.bitcast_convert_type(
        jnp.asarray(new_value, jnp.float32), jnp.int32)
    noise = jax.random.normal(jax.random.key(42), (), dtype=jnp.float32)
    hn_bits = lax.bitcast_convert_type(noise * jnp.float32(0.5), jnp.int32)
    scal = jnp.stack([idx, nv_bits, hn_bits]).reshape(1, 3)
    upd, thr = _call(scal, history.reshape(_ROWS, _COLS))
    return upd.reshape(_SIZE), thr[0, 0]


# speedup vs baseline: 5.8025x; 1.0710x over previous
import jax
import jax.numpy as jnp
from jax import lax
from jax.experimental import pallas as pl
from jax.experimental.pallas import tpu as pltpu

_SIZE = 65536
_VALID = 40001


def _body(ptr_ref, nv_ref, hn_ref, hist_ref, out_ref, thr_ref):
    h = hist_ref[...]
    idx = ptr_ref[0, 0] % _SIZE
    nv = nv_ref[0, 0]
    halfnoise = hn_ref[0, 0]
    flat = lax.broadcasted_iota(jnp.int32, (_SIZE,), 0)
    upd = jnp.where(flat == idx, nv, h)
    out_ref[...] = upd
    vm = jnp.where(flat < _VALID, upd, 0.0)
    s = jnp.sum(vm)
    q = jnp.sum(vm * vm)
    inv_n = jnp.float32(1.0 / _VALID)
    mean = s * inv_n
    var = jnp.maximum(q * inv_n - mean * mean, 0.0)
    std = jnp.sqrt(var)
    thr_ref[0, 0] = mean + halfnoise * std


_call = pl.pallas_call(
    _body,
    out_shape=(
        jax.ShapeDtypeStruct((_SIZE,), jnp.float32),
        jax.ShapeDtypeStruct((1, 1), jnp.float32),
    ),
    in_specs=[
        pl.BlockSpec(memory_space=pltpu.SMEM),
        pl.BlockSpec(memory_space=pltpu.SMEM),
        pl.BlockSpec(memory_space=pltpu.SMEM),
        pl.BlockSpec(memory_space=pltpu.VMEM),
    ],
    out_specs=(
        pl.BlockSpec(memory_space=pltpu.VMEM),
        pl.BlockSpec(memory_space=pltpu.SMEM),
    ),
)


@jax.jit
def kernel(history, new_value, pointer):
    ptr = jnp.asarray(pointer, jnp.int32).reshape(1, 1)
    nv = jnp.asarray(new_value, jnp.float32).reshape(1, 1)
    noise = jax.random.normal(jax.random.key(42), (), dtype=jnp.float32)
    hn = (noise * jnp.float32(0.5)).reshape(1, 1)
    upd, thr = _call(ptr, nv, hn, history)
    return upd, thr[0, 0]


# speedup vs baseline: 6.1440x; 1.0589x over previous
import jax
import jax.numpy as jnp
from jax import lax
from jax.experimental import pallas as pl
from jax.experimental.pallas import tpu as pltpu

_SIZE = 65536
_VALID = 40001


_FULL = 39936
_TAIL = _VALID - _FULL


def _body(ptr_ref, nv_ref, hn_ref, hist_ref, out_ref, thr_ref):
    idx = ptr_ref[0, 0] % _SIZE
    nv = nv_ref[0, 0]
    halfnoise = hn_ref[0, 0]
    out_ref[...] = hist_ref[...]
    base = pl.multiple_of((idx // 128) * 128, 128)
    off = idx % 128
    blk = out_ref[pl.ds(base, 128)]
    sel = lax.broadcasted_iota(jnp.int32, (128,), 0) == off
    out_ref[pl.ds(base, 128)] = jnp.where(sel, nv, blk)
    a = out_ref[pl.ds(0, _FULL)]
    tail = out_ref[pl.ds(_FULL, 1024)]
    tmask = lax.broadcasted_iota(jnp.int32, (1024,), 0) < _TAIL
    tm = jnp.where(tmask, tail, 0.0)
    s = jnp.sum(a) + jnp.sum(tm)
    q = jnp.sum(a * a) + jnp.sum(tm * tm)
    inv_n = jnp.float32(1.0 / _VALID)
    mean = s * inv_n
    var = jnp.maximum(q * inv_n - mean * mean, 0.0)
    std = jnp.sqrt(var)
    thr_ref[0, 0] = mean + halfnoise * std


_call = pl.pallas_call(
    _body,
    out_shape=(
        jax.ShapeDtypeStruct((_SIZE,), jnp.float32),
        jax.ShapeDtypeStruct((1, 1), jnp.float32),
    ),
    in_specs=[
        pl.BlockSpec(memory_space=pltpu.SMEM),
        pl.BlockSpec(memory_space=pltpu.SMEM),
        pl.BlockSpec(memory_space=pltpu.SMEM),
        pl.BlockSpec(memory_space=pltpu.VMEM),
    ],
    out_specs=(
        pl.BlockSpec(memory_space=pltpu.VMEM),
        pl.BlockSpec(memory_space=pltpu.SMEM),
    ),
)


@jax.jit
def kernel(history, new_value, pointer):
    ptr = jnp.asarray(pointer, jnp.int32).reshape(1, 1)
    nv = jnp.asarray(new_value, jnp.float32).reshape(1, 1)
    noise = jax.random.normal(jax.random.key(42), (), dtype=jnp.float32)
    hn = (noise * jnp.float32(0.5)).reshape(1, 1)
    upd, thr = _call(ptr, nv, hn, history)
    return upd, thr[0, 0]
